# Initial kernel scaffold; baseline (speedup 1.0000x reference)
import jax
import jax.numpy as jnp
from jax.experimental import pallas as pl


def kernel(x, edge_index, edge_weight, W_diff, b_diff, W_ih, W_hh, b_ih, b_hh, W_mu, b_mu, W_sigma, b_sigma, W_pi, b_pi):
    raise NotImplementedError("write your pallas kernel here")


# speedup vs baseline: 32.9688x; 32.9688x over previous
import functools

import jax
import jax.numpy as jnp
from jax import lax
from jax.experimental import pallas as pl
from jax.experimental.pallas import tpu as pltpu
from jax.experimental.pallas import tpu_sc as plsc

_B = 16
_T = 2048
_F = 16
_E = 32768
_H = 128
_M = 8
_D = 16
_L = 16
_EC = 4096
_CT = 128


def _sc_diffusion(x, edge_index, edge_weight):
    mesh = plsc.VectorSubcoreMesh(core_axis_name="c", subcore_axis_name="s",
                                  num_cores=2, num_subcores=16)
    nchunk = _E // _EC
    ngrp = _EC // _L

    @functools.partial(
        pl.kernel,
        out_type=jax.ShapeDtypeStruct((_B, 4, _T, _F), jnp.float32),
        mesh=mesh,
        scratch_types=[
            pltpu.VMEM((_T, _F), jnp.float32),
            pltpu.VMEM((_T, _F), jnp.float32),
            pltpu.VMEM((_T, _F), jnp.float32),
            pltpu.VMEM((_T,), jnp.float32),
            pltpu.VMEM((_EC,), jnp.int32),
            pltpu.VMEM((_EC,), jnp.int32),
            pltpu.VMEM((_EC,), jnp.float32),
        ],
        compiler_params=pltpu.CompilerParams(needs_layout_passes=False,
                                             use_tc_tiling_on_sc=False),
    )
    def sc_kernel(x_hbm, ei_hbm, ew_hbm, out_hbm, xv, o1, o2, deg, gch, sch, wch):
        c = lax.axis_index("c")
        s = lax.axis_index("s")
        b = s
        d = c
        od = 1 - d

        zrow = jnp.zeros((_L,), jnp.float32)

        def zero_rows(i, _):
            o1[i] = zrow
            o2[i] = zrow
            return 0

        lax.fori_loop(0, _T, zero_rows, 0)

        def zero_deg(i, _):
            deg[pl.ds(i * _L, _L)] = zrow
            return 0

        lax.fori_loop(0, _T // _L, zero_deg, 0)

        pltpu.sync_copy(x_hbm.at[b], xv)

        def deg_chunk(ch, _):
            pltpu.sync_copy(ei_hbm.at[b, d, pl.ds(ch * _EC, _EC)], gch)
            pltpu.sync_copy(ew_hbm.at[b, pl.ds(ch * _EC, _EC)], wch)

            def grp(i, _):
                gi = gch[pl.ds(i * _L, _L)]
                w = wch[pl.ds(i * _L, _L)]
                plsc.addupdate_scatter(deg, [gi], w)
                return 0

            lax.fori_loop(0, ngrp, grp, 0)
            return 0

        lax.fori_loop(0, nchunk, deg_chunk, 0)

        def prop(src_ref, tgt_ref):
            def chunk_body(ch, _):
                pltpu.sync_copy(ei_hbm.at[b, d, pl.ds(ch * _EC, _EC)], gch)
                pltpu.sync_copy(ei_hbm.at[b, od, pl.ds(ch * _EC, _EC)], sch)
                pltpu.sync_copy(ew_hbm.at[b, pl.ds(ch * _EC, _EC)], wch)

                def grp(i, _):
                    gi = gch[pl.ds(i * _L, _L)]
                    si = sch[pl.ds(i * _L, _L)]
                    w = wch[pl.ds(i * _L, _L)]
                    wn = w / plsc.load_gather(deg, [gi])
                    for f in range(_F):
                        fs = jnp.full((_L,), f, jnp.int32)
                        v = plsc.load_gather(src_ref, [gi, fs])
                        plsc.addupdate_scatter(tgt_ref, [si, fs], v * wn)
                    return 0

                lax.fori_loop(0, ngrp, grp, 0)
                return 0

            lax.fori_loop(0, nchunk, chunk_body, 0)

        prop(xv, o1)
        prop(o1, o2)

        pltpu.sync_copy(o1, out_hbm.at[b, 2 * d])
        pltpu.sync_copy(o2, out_hbm.at[b, 2 * d + 1])

    return sc_kernel(x, edge_index, edge_weight)


def _tc_body(z_ref, x_ref, wdiff_ref, bdiff_ref, wiht_ref, whht_ref, bg_ref,
             wmu_ref, bmu_ref, wsig_ref, bsig_ref, wpi_ref, bpi_ref,
             mu_ref, sig_ref, pi_ref, h_s, c_s, xw_s, hs_s):
    i = pl.program_id(0)

    @pl.when(i == 0)
    def _():
        h_s[...] = jnp.zeros_like(h_s)
        c_s[...] = jnp.zeros_like(c_s)

    wiht = wiht_ref[...]
    wz = jnp.dot(wdiff_ref[...], wiht[:_H, :],
                 preferred_element_type=jnp.float32)
    bias = (jnp.dot(bdiff_ref[...], wiht[:_H, :],
                    preferred_element_type=jnp.float32)
            + bg_ref[...])

    z = z_ref[...].reshape(_CT * _B, 4 * _F)
    xx = x_ref[...].reshape(_CT * _B, _F)
    xw = (jnp.dot(z, wz, preferred_element_type=jnp.float32)
          + jnp.dot(xx, wiht[_H:, :], preferred_element_type=jnp.float32)
          + bias)
    xw_s[...] = xw.reshape(_CT, _B, 4 * _H)

    whht = whht_ref[...]

    def step(t, carry):
        h, c = carry
        g = xw_s[t] + jnp.dot(h, whht, preferred_element_type=jnp.float32)
        ig = jax.nn.sigmoid(g[:, 0 * _H:1 * _H])
        fg = jax.nn.sigmoid(g[:, 1 * _H:2 * _H])
        gg = jnp.tanh(g[:, 2 * _H:3 * _H])
        og = jax.nn.sigmoid(g[:, 3 * _H:4 * _H])
        c2 = fg * c + ig * gg
        h2 = og * jnp.tanh(c2)
        hs_s[t] = h2
        return h2, c2

    hN, cN = lax.fori_loop(0, _CT, step, (h_s[...], c_s[...]))
    h_s[...] = hN
    c_s[...] = cN

    hs = hs_s[...].reshape(_CT * _B, _H)
    mu = jnp.dot(hs, wmu_ref[...], preferred_element_type=jnp.float32) + bmu_ref[...]
    mu_ref[...] = mu.reshape(_CT, _B, _M * _D)
    sg = jnp.exp(jnp.dot(hs, wsig_ref[...], preferred_element_type=jnp.float32)
                 + bsig_ref[...])
    sig_ref[...] = sg.reshape(_CT, _B, _M * _D)
    logits = jnp.dot(hs, wpi_ref[...], preferred_element_type=jnp.float32) + bpi_ref[...]
    logits = logits - jnp.max(logits, axis=-1, keepdims=True)
    e = jnp.exp(logits)
    pi = e / jnp.sum(e, axis=-1, keepdims=True)
    pi_ref[...] = pi.reshape(_CT, _B, _M)


def _tc_lstm(z_t, x_t, W_diff, b_diff, W_ih_T, W_hh_T, b_gate,
             W_mu, b_mu, W_sigma, b_sigma, W_pi, b_pi):
    ng = _T // _CT

    def full(a):
        return pl.BlockSpec(a.shape, lambda i: (0,) * a.ndim)

    in_specs = [
            pl.BlockSpec((_CT, _B, 4 * _F), lambda i: (i, 0, 0)),
            pl.BlockSpec((_CT, _B, _F), lambda i: (i, 0, 0)),
            full(W_diff), full(b_diff), full(W_ih_T), full(W_hh_T),
            full(b_gate), full(W_mu), full(b_mu), full(W_sigma),
            full(b_sigma), full(W_pi), full(b_pi),
    ]
    out_specs = [
        pl.BlockSpec((_CT, _B, _M * _D), lambda i: (i, 0, 0)),
        pl.BlockSpec((_CT, _B, _M * _D), lambda i: (i, 0, 0)),
        pl.BlockSpec((_CT, _B, _M), lambda i: (i, 0, 0)),
    ]
    out_shape = [
        jax.ShapeDtypeStruct((_T, _B, _M * _D), jnp.float32),
        jax.ShapeDtypeStruct((_T, _B, _M * _D), jnp.float32),
        jax.ShapeDtypeStruct((_T, _B, _M), jnp.float32),
    ]
    return pl.pallas_call(
        _tc_body,
        grid=(ng,),
        in_specs=in_specs,
        out_specs=out_specs,
        out_shape=out_shape,
        scratch_shapes=[
            pltpu.VMEM((_B, _H), jnp.float32),
            pltpu.VMEM((_B, _H), jnp.float32),
            pltpu.VMEM((_CT, _B, 4 * _H), jnp.float32),
            pltpu.VMEM((_CT, _B, _H), jnp.float32),
        ],
    )(z_t, x_t, W_diff, b_diff, W_ih_T, W_hh_T, b_gate,
      W_mu, b_mu, W_sigma, b_sigma, W_pi, b_pi)


def kernel(x, edge_index, edge_weight, W_diff, b_diff, W_ih, W_hh, b_ih, b_hh,
           W_mu, b_mu, W_sigma, b_sigma, W_pi, b_pi):
    out4 = _sc_diffusion(x, edge_index, edge_weight)
    z_t = out4.transpose(2, 0, 1, 3).reshape(_T, _B, 4 * _F)
    x_t = x.transpose(1, 0, 2)

    mu_t, sig_t, pi_t = _tc_lstm(
        z_t, x_t, W_diff, b_diff.reshape(1, _H),
        W_ih.T, W_hh.T, (b_ih + b_hh).reshape(1, 4 * _H),
        W_mu, b_mu.reshape(1, _M * _D), W_sigma, b_sigma.reshape(1, _M * _D),
        W_pi, b_pi.reshape(1, _M))

    mu = mu_t.transpose(1, 0, 2).reshape(_B, _T, _M, _D)
    sigma = sig_t.transpose(1, 0, 2).reshape(_B, _T, _M, _D)
    pi = pi_t.transpose(1, 0, 2)
    return (mu, sigma, pi)


# speedup vs baseline: 46.2730x; 1.4035x over previous
import functools

import jax
import jax.numpy as jnp
from jax import lax
from jax.experimental import pallas as pl
from jax.experimental.pallas import tpu as pltpu
from jax.experimental.pallas import tpu_sc as plsc

_B = 16
_T = 2048
_F = 16
_E = 32768
_H = 128
_M = 8
_D = 16
_L = 16
_EC = 4096
_CT = 128


def _sc_diffusion(x, edge_index, edge_weight):
    mesh = plsc.VectorSubcoreMesh(core_axis_name="c", subcore_axis_name="s",
                                  num_cores=2, num_subcores=16)
    nchunk = _E // _EC
    ngrp = _EC // _L

    @functools.partial(
        pl.kernel,
        out_type=jax.ShapeDtypeStruct((_B, 4, _T, _F), jnp.float32),
        mesh=mesh,
        scratch_types=[
            pltpu.VMEM((_T, _F), jnp.float32),
            pltpu.VMEM((_T, _F), jnp.float32),
            pltpu.VMEM((_T, _F), jnp.float32),
            pltpu.VMEM((_T,), jnp.float32),
            pltpu.VMEM((_EC,), jnp.int32),
            pltpu.VMEM((_EC,), jnp.int32),
            pltpu.VMEM((_EC,), jnp.float32),
        ],
        compiler_params=pltpu.CompilerParams(needs_layout_passes=False,
                                             use_tc_tiling_on_sc=False),
    )
    def sc_kernel(x_hbm, ei_hbm, ew_hbm, out_hbm, xv, o1, o2, deg, gch, sch, wch):
        c = lax.axis_index("c")
        s = lax.axis_index("s")
        b = s
        d = c
        od = 1 - d

        zrow = jnp.zeros((_L,), jnp.float32)

        def zero_rows(i, _):
            o1[i] = zrow
            o2[i] = zrow
            return 0

        lax.fori_loop(0, _T, zero_rows, 0)

        def zero_deg(i, _):
            deg[pl.ds(i * _L, _L)] = zrow
            return 0

        lax.fori_loop(0, _T // _L, zero_deg, 0)

        pltpu.sync_copy(x_hbm.at[b], xv)

        def deg_chunk(ch, _):
            pltpu.sync_copy(ei_hbm.at[b, d, pl.ds(ch * _EC, _EC)], gch)
            pltpu.sync_copy(ew_hbm.at[b, pl.ds(ch * _EC, _EC)], wch)

            @plsc.parallel_loop(0, ngrp, 1, unroll=4)
            def grp(i):
                gi = gch[pl.ds(i * _L, _L)]
                w = wch[pl.ds(i * _L, _L)]
                plsc.addupdate_scatter(deg, [gi], w)
            return 0

        lax.fori_loop(0, nchunk, deg_chunk, 0)

        def prop(src_ref, tgt_ref):
            def chunk_body(ch, _):
                pltpu.sync_copy(ei_hbm.at[b, d, pl.ds(ch * _EC, _EC)], gch)
                pltpu.sync_copy(ei_hbm.at[b, od, pl.ds(ch * _EC, _EC)], sch)
                pltpu.sync_copy(ew_hbm.at[b, pl.ds(ch * _EC, _EC)], wch)

                @plsc.parallel_loop(0, ngrp, 1, unroll=2)
                def grp(i):
                    gi = gch[pl.ds(i * _L, _L)]
                    si = sch[pl.ds(i * _L, _L)]
                    w = wch[pl.ds(i * _L, _L)]
                    wn = w / plsc.load_gather(deg, [gi])
                    vals = [plsc.load_gather(src_ref,
                                             [gi, jnp.full((_L,), f, jnp.int32)])
                            for f in range(_F)]
                    for f in range(_F):
                        plsc.addupdate_scatter(
                            tgt_ref, [si, jnp.full((_L,), f, jnp.int32)],
                            vals[f] * wn)
                return 0

            lax.fori_loop(0, nchunk, chunk_body, 0)

        prop(xv, o1)
        prop(o1, o2)

        pltpu.sync_copy(o1, out_hbm.at[b, 2 * d])
        pltpu.sync_copy(o2, out_hbm.at[b, 2 * d + 1])

    return sc_kernel(x, edge_index, edge_weight)


def _tc_body(z_ref, x_ref, wdiff_ref, bdiff_ref, wiht_ref, whht_ref, bg_ref,
             wmu_ref, bmu_ref, wsig_ref, bsig_ref, wpi_ref, bpi_ref,
             mu_ref, sig_ref, pi_ref, h_s, c_s, xw_s, hs_s):
    i = pl.program_id(0)

    @pl.when(i == 0)
    def _():
        h_s[...] = jnp.zeros_like(h_s)
        c_s[...] = jnp.zeros_like(c_s)

    wiht = wiht_ref[...]
    wz = jnp.dot(wdiff_ref[...], wiht[:_H, :],
                 preferred_element_type=jnp.float32)
    bias = (jnp.dot(bdiff_ref[...], wiht[:_H, :],
                    preferred_element_type=jnp.float32)
            + bg_ref[...])

    z = z_ref[...].reshape(_CT * _B, 4 * _F)
    xx = x_ref[...].reshape(_CT * _B, _F)
    xw = (jnp.dot(z, wz, preferred_element_type=jnp.float32)
          + jnp.dot(xx, wiht[_H:, :], preferred_element_type=jnp.float32)
          + bias)
    xw_s[...] = xw.reshape(_CT, _B, 4 * _H)

    whht = whht_ref[...]

    def step(t, carry):
        h, c = carry
        g = xw_s[t] + jnp.dot(h, whht, preferred_element_type=jnp.float32)
        ig = jax.nn.sigmoid(g[:, 0 * _H:1 * _H])
        fg = jax.nn.sigmoid(g[:, 1 * _H:2 * _H])
        gg = jnp.tanh(g[:, 2 * _H:3 * _H])
        og = jax.nn.sigmoid(g[:, 3 * _H:4 * _H])
        c2 = fg * c + ig * gg
        h2 = og * jnp.tanh(c2)
        hs_s[t] = h2
        return h2, c2

    hN, cN = lax.fori_loop(0, _CT, step, (h_s[...], c_s[...]))
    h_s[...] = hN
    c_s[...] = cN

    hs = hs_s[...].reshape(_CT * _B, _H)
    mu = jnp.dot(hs, wmu_ref[...], preferred_element_type=jnp.float32) + bmu_ref[...]
    mu_ref[...] = mu.reshape(_CT, _B, _M * _D)
    sg = jnp.exp(jnp.dot(hs, wsig_ref[...], preferred_element_type=jnp.float32)
                 + bsig_ref[...])
    sig_ref[...] = sg.reshape(_CT, _B, _M * _D)
    logits = jnp.dot(hs, wpi_ref[...], preferred_element_type=jnp.float32) + bpi_ref[...]
    logits = logits - jnp.max(logits, axis=-1, keepdims=True)
    e = jnp.exp(logits)
    pi = e / jnp.sum(e, axis=-1, keepdims=True)
    pi_ref[...] = pi.reshape(_CT, _B, _M)


def _tc_lstm(z_t, x_t, W_diff, b_diff, W_ih_T, W_hh_T, b_gate,
             W_mu, b_mu, W_sigma, b_sigma, W_pi, b_pi):
    ng = _T // _CT

    def full(a):
        return pl.BlockSpec(a.shape, lambda i: (0,) * a.ndim)

    in_specs = [
            pl.BlockSpec((_CT, _B, 4 * _F), lambda i: (i, 0, 0)),
            pl.BlockSpec((_CT, _B, _F), lambda i: (i, 0, 0)),
            full(W_diff), full(b_diff), full(W_ih_T), full(W_hh_T),
            full(b_gate), full(W_mu), full(b_mu), full(W_sigma),
            full(b_sigma), full(W_pi), full(b_pi),
    ]
    out_specs = [
        pl.BlockSpec((_CT, _B, _M * _D), lambda i: (i, 0, 0)),
        pl.BlockSpec((_CT, _B, _M * _D), lambda i: (i, 0, 0)),
        pl.BlockSpec((_CT, _B, _M), lambda i: (i, 0, 0)),
    ]
    out_shape = [
        jax.ShapeDtypeStruct((_T, _B, _M * _D), jnp.float32),
        jax.ShapeDtypeStruct((_T, _B, _M * _D), jnp.float32),
        jax.ShapeDtypeStruct((_T, _B, _M), jnp.float32),
    ]
    return pl.pallas_call(
        _tc_body,
        grid=(ng,),
        in_specs=in_specs,
        out_specs=out_specs,
        out_shape=out_shape,
        scratch_shapes=[
            pltpu.VMEM((_B, _H), jnp.float32),
            pltpu.VMEM((_B, _H), jnp.float32),
            pltpu.VMEM((_CT, _B, 4 * _H), jnp.float32),
            pltpu.VMEM((_CT, _B, _H), jnp.float32),
        ],
    )(z_t, x_t, W_diff, b_diff, W_ih_T, W_hh_T, b_gate,
      W_mu, b_mu, W_sigma, b_sigma, W_pi, b_pi)


def kernel(x, edge_index, edge_weight, W_diff, b_diff, W_ih, W_hh, b_ih, b_hh,
           W_mu, b_mu, W_sigma, b_sigma, W_pi, b_pi):
    out4 = _sc_diffusion(x, edge_index, edge_weight)
    z_t = out4.transpose(2, 0, 1, 3).reshape(_T, _B, 4 * _F)
    x_t = x.transpose(1, 0, 2)

    mu_t, sig_t, pi_t = _tc_lstm(
        z_t, x_t, W_diff, b_diff.reshape(1, _H),
        W_ih.T, W_hh.T, (b_ih + b_hh).reshape(1, 4 * _H),
        W_mu, b_mu.reshape(1, _M * _D), W_sigma, b_sigma.reshape(1, _M * _D),
        W_pi, b_pi.reshape(1, _M))

    mu = mu_t.transpose(1, 0, 2).reshape(_B, _T, _M, _D)
    sigma = sig_t.transpose(1, 0, 2).reshape(_B, _T, _M, _D)
    pi = pi_t.transpose(1, 0, 2)
    return (mu, sigma, pi)


# speedup vs baseline: 48.2671x; 1.0431x over previous
import functools

import jax
import jax.numpy as jnp
from jax import lax
from jax.experimental import pallas as pl
from jax.experimental.pallas import tpu as pltpu
from jax.experimental.pallas import tpu_sc as plsc

_B = 16
_T = 2048
_F = 16
_E = 32768
_H = 128
_M = 8
_D = 16
_L = 16
_EC = 4096
_CT = 128


def _sc_diffusion(x, edge_index, edge_weight):
    mesh = plsc.VectorSubcoreMesh(core_axis_name="c", subcore_axis_name="s",
                                  num_cores=2, num_subcores=16)
    nchunk = _E // _EC
    ngrp = _EC // _L

    @functools.partial(
        pl.kernel,
        out_type=jax.ShapeDtypeStruct((_B, 4, _T, _F), jnp.float32),
        mesh=mesh,
        scratch_types=[
            pltpu.VMEM((_T, _F), jnp.float32),
            pltpu.VMEM((_T, _F), jnp.float32),
            pltpu.VMEM((_T, _F), jnp.float32),
            pltpu.VMEM((_T,), jnp.float32),
            pltpu.VMEM((2, _EC), jnp.int32),
            pltpu.VMEM((2, _EC), jnp.int32),
            pltpu.VMEM((2, _EC), jnp.float32),
            pltpu.SemaphoreType.DMA,
            pltpu.SemaphoreType.DMA,
        ],
        compiler_params=pltpu.CompilerParams(needs_layout_passes=False,
                                             use_tc_tiling_on_sc=False),
    )
    def sc_kernel(x_hbm, ei_hbm, ew_hbm, out_hbm, xv, o1, o2, deg,
                  gch, sch, wch, esem, xsem):
        c = lax.axis_index("c")
        s = lax.axis_index("s")
        b = s
        d = c
        od = 1 - d

        pltpu.async_copy(x_hbm.at[b], xv, xsem)

        zrow = jnp.zeros((_L,), jnp.float32)

        def zero_rows(i, _):
            o1[i] = zrow
            o2[i] = zrow
            return 0

        lax.fori_loop(0, _T, zero_rows, 0)

        def zero_deg(i, _):
            deg[pl.ds(i * _L, _L)] = zrow
            return 0

        lax.fori_loop(0, _T // _L, zero_deg, 0)

        def issue(ch, slot, with_sidx):
            pltpu.async_copy(ei_hbm.at[b, d, pl.ds(ch * _EC, _EC)],
                             gch.at[slot], esem)
            if with_sidx:
                pltpu.async_copy(ei_hbm.at[b, od, pl.ds(ch * _EC, _EC)],
                                 sch.at[slot], esem)
            pltpu.async_copy(ew_hbm.at[b, pl.ds(ch * _EC, _EC)],
                             wch.at[slot], esem)

        def wait(ch, slot, with_sidx):
            pltpu.make_async_copy(ei_hbm.at[b, d, pl.ds(ch * _EC, _EC)],
                                  gch.at[slot], esem).wait()
            if with_sidx:
                pltpu.make_async_copy(ei_hbm.at[b, od, pl.ds(ch * _EC, _EC)],
                                      sch.at[slot], esem).wait()
            pltpu.make_async_copy(ew_hbm.at[b, pl.ds(ch * _EC, _EC)],
                                  wch.at[slot], esem).wait()

        def stream_chunks(with_sidx, process):
            issue(0, 0, with_sidx)

            def chunk_body(ch, _):
                slot = lax.rem(ch, 2)
                wait(ch, slot, with_sidx)

                @pl.when(ch + 1 < nchunk)
                def _():
                    issue(ch + 1, 1 - slot, with_sidx)

                process(slot)
                return 0

            lax.fori_loop(0, nchunk, chunk_body, 0)

        def deg_process(slot):
            @plsc.parallel_loop(0, ngrp, 1, unroll=4)
            def grp(i):
                gi = gch[slot, pl.ds(i * _L, _L)]
                w = wch[slot, pl.ds(i * _L, _L)]
                plsc.addupdate_scatter(deg, [gi], w)

        stream_chunks(False, deg_process)
        pltpu.make_async_copy(x_hbm.at[b], xv, xsem).wait()

        def prop(src_ref, tgt_ref):
            def process(slot):
                @plsc.parallel_loop(0, ngrp, 1, unroll=2)
                def grp(i):
                    gi = gch[slot, pl.ds(i * _L, _L)]
                    si = sch[slot, pl.ds(i * _L, _L)]
                    w = wch[slot, pl.ds(i * _L, _L)]
                    wn = w / plsc.load_gather(deg, [gi])
                    vals = [plsc.load_gather(src_ref,
                                             [gi, jnp.full((_L,), f, jnp.int32)])
                            for f in range(_F)]
                    for f in range(_F):
                        plsc.addupdate_scatter(
                            tgt_ref, [si, jnp.full((_L,), f, jnp.int32)],
                            vals[f] * wn)

            stream_chunks(True, process)

        prop(xv, o1)
        prop(o1, o2)

        pltpu.sync_copy(o1, out_hbm.at[b, 2 * d])
        pltpu.sync_copy(o2, out_hbm.at[b, 2 * d + 1])

    return sc_kernel(x, edge_index, edge_weight)


def _tc_body(z_ref, x_ref, wdiff_ref, bdiff_ref, wiht_ref, whht_ref, bg_ref,
             wmu_ref, bmu_ref, wsig_ref, bsig_ref, wpi_ref, bpi_ref,
             mu_ref, sig_ref, pi_ref, h_s, c_s, xw_s, hs_s):
    i = pl.program_id(0)

    @pl.when(i == 0)
    def _():
        h_s[...] = jnp.zeros_like(h_s)
        c_s[...] = jnp.zeros_like(c_s)

    wiht = wiht_ref[...]
    wz = jnp.dot(wdiff_ref[...], wiht[:_H, :],
                 preferred_element_type=jnp.float32)
    bias = (jnp.dot(bdiff_ref[...], wiht[:_H, :],
                    preferred_element_type=jnp.float32)
            + bg_ref[...])

    z = z_ref[...].reshape(_CT * _B, 4 * _F)
    xx = x_ref[...].reshape(_CT * _B, _F)
    xw = (jnp.dot(z, wz, preferred_element_type=jnp.float32)
          + jnp.dot(xx, wiht[_H:, :], preferred_element_type=jnp.float32)
          + bias)
    xw_s[...] = xw.reshape(_CT, _B, 4 * _H)

    whht = whht_ref[...]

    def step(t, carry):
        h, c = carry
        g = xw_s[t] + jnp.dot(h, whht, preferred_element_type=jnp.float32)
        ig = jax.nn.sigmoid(g[:, 0 * _H:1 * _H])
        fg = jax.nn.sigmoid(g[:, 1 * _H:2 * _H])
        gg = jnp.tanh(g[:, 2 * _H:3 * _H])
        og = jax.nn.sigmoid(g[:, 3 * _H:4 * _H])
        c2 = fg * c + ig * gg
        h2 = og * jnp.tanh(c2)
        hs_s[t] = h2
        return h2, c2

    hN, cN = lax.fori_loop(0, _CT, step, (h_s[...], c_s[...]))
    h_s[...] = hN
    c_s[...] = cN

    hs = hs_s[...].reshape(_CT * _B, _H)
    mu = jnp.dot(hs, wmu_ref[...], preferred_element_type=jnp.float32) + bmu_ref[...]
    mu_ref[...] = mu.reshape(_CT, _B, _M * _D)
    sg = jnp.exp(jnp.dot(hs, wsig_ref[...], preferred_element_type=jnp.float32)
                 + bsig_ref[...])
    sig_ref[...] = sg.reshape(_CT, _B, _M * _D)
    logits = jnp.dot(hs, wpi_ref[...], preferred_element_type=jnp.float32) + bpi_ref[...]
    logits = logits - jnp.max(logits, axis=-1, keepdims=True)
    e = jnp.exp(logits)
    pi = e / jnp.sum(e, axis=-1, keepdims=True)
    pi_ref[...] = pi.reshape(_CT, _B, _M)


def _tc_lstm(z_t, x_t, W_diff, b_diff, W_ih_T, W_hh_T, b_gate,
             W_mu, b_mu, W_sigma, b_sigma, W_pi, b_pi):
    ng = _T // _CT

    def full(a):
        return pl.BlockSpec(a.shape, lambda i: (0,) * a.ndim)

    in_specs = [
            pl.BlockSpec((_CT, _B, 4 * _F), lambda i: (i, 0, 0)),
            pl.BlockSpec((_CT, _B, _F), lambda i: (i, 0, 0)),
            full(W_diff), full(b_diff), full(W_ih_T), full(W_hh_T),
            full(b_gate), full(W_mu), full(b_mu), full(W_sigma),
            full(b_sigma), full(W_pi), full(b_pi),
    ]
    out_specs = [
        pl.BlockSpec((_CT, _B, _M * _D), lambda i: (i, 0, 0)),
        pl.BlockSpec((_CT, _B, _M * _D), lambda i: (i, 0, 0)),
        pl.BlockSpec((_CT, _B, _M), lambda i: (i, 0, 0)),
    ]
    out_shape = [
        jax.ShapeDtypeStruct((_T, _B, _M * _D), jnp.float32),
        jax.ShapeDtypeStruct((_T, _B, _M * _D), jnp.float32),
        jax.ShapeDtypeStruct((_T, _B, _M), jnp.float32),
    ]
    return pl.pallas_call(
        _tc_body,
        grid=(ng,),
        in_specs=in_specs,
        out_specs=out_specs,
        out_shape=out_shape,
        scratch_shapes=[
            pltpu.VMEM((_B, _H), jnp.float32),
            pltpu.VMEM((_B, _H), jnp.float32),
            pltpu.VMEM((_CT, _B, 4 * _H), jnp.float32),
            pltpu.VMEM((_CT, _B, _H), jnp.float32),
        ],
    )(z_t, x_t, W_diff, b_diff, W_ih_T, W_hh_T, b_gate,
      W_mu, b_mu, W_sigma, b_sigma, W_pi, b_pi)


def kernel(x, edge_index, edge_weight, W_diff, b_diff, W_ih, W_hh, b_ih, b_hh,
           W_mu, b_mu, W_sigma, b_sigma, W_pi, b_pi):
    out4 = _sc_diffusion(x, edge_index, edge_weight)
    z_t = out4.transpose(2, 0, 1, 3).reshape(_T, _B, 4 * _F)
    x_t = x.transpose(1, 0, 2)

    mu_t, sig_t, pi_t = _tc_lstm(
        z_t, x_t, W_diff, b_diff.reshape(1, _H),
        W_ih.T, W_hh.T, (b_ih + b_hh).reshape(1, 4 * _H),
        W_mu, b_mu.reshape(1, _M * _D), W_sigma, b_sigma.reshape(1, _M * _D),
        W_pi, b_pi.reshape(1, _M))

    mu = mu_t.transpose(1, 0, 2).reshape(_B, _T, _M, _D)
    sigma = sig_t.transpose(1, 0, 2).reshape(_B, _T, _M, _D)
    pi = pi_t.transpose(1, 0, 2)
    return (mu, sigma, pi)
